# baseline (speedup 1.0000x reference)
import functools

import jax
import jax.numpy as jnp
from jax import lax
from jax.experimental import pallas as pl
from jax.experimental.pallas import tpu as pltpu
from jax.experimental.pallas import tpu_sc as plsc

N_NODES = 10000
N_EDGES = 320000
D = 128
L = 16

NC = 2
NS = 16
NW = NC * NS

NPAD = 10240
EDGES_PER_TILE = 10240
CHUNK = 32
KBUF = 4
GROUP = KBUF * CHUNK
NGROUPS = EDGES_PER_TILE // GROUP
NPAIRS = NGROUPS // 2
NCHUNKS = EDGES_PER_TILE // CHUNK
DEGB = 512
NDEGG = EDGES_PER_TILE // DEGB
ZROWS = NPAD // NS


def _sc_segment_sum(x, src, dst, dst2, zeros_feat, zeros_deg):
    mesh = plsc.VectorSubcoreMesh(
        core_axis_name="c", subcore_axis_name="s", num_cores=NC, num_subcores=NS
    )

    @functools.partial(
        pl.kernel,
        mesh=mesh,
        compiler_params=pltpu.CompilerParams(needs_layout_passes=False),
        out_type=[
            jax.ShapeDtypeStruct((NC * NPAD, D), jnp.float32),
            jax.ShapeDtypeStruct((NW * NPAD,), jnp.float32),
        ],
        scratch_types=[
            pltpu.VMEM((2 * GROUP,), jnp.int32),
            pltpu.VMEM((2 * KBUF, CHUNK), jnp.int32),
            pltpu.VMEM((2, KBUF, CHUNK, D), jnp.float32),
            pltpu.VMEM((DEGB,), jnp.int32),
            pltpu.VMEM((NPAD,), jnp.float32),
            pltpu.VMEM_SHARED((NPAD, D), jnp.float32),
            pltpu.SemaphoreType.DMA,
            pltpu.SemaphoreType.DMA,
        ],
    )
    def seg_kernel(x_hbm, src_hbm, dst_hbm, dst2_hbm, zf_hbm, zd_hbm,
                   acc_out, deg_out,
                   sidx_v, didx_v, rows_v, degidx_v, deg_v, acc_s,
                   gsem, ssem):
        cid = lax.axis_index("c")
        sid = lax.axis_index("s")
        wid = sid * NC + cid
        zbase = sid * ZROWS
        ebase = wid * EDGES_PER_TILE

        pltpu.sync_copy(zf_hbm.at[pl.ds(zbase, ZROWS)], acc_s.at[pl.ds(zbase, ZROWS)])
        pltpu.sync_copy(zd_hbm, deg_v)
        plsc.subcore_barrier()

        def gather_group(slot):
            return [
                pltpu.async_copy(
                    x_hbm.at[sidx_v.at[pl.ds(slot * GROUP + b * CHUNK, CHUNK)]],
                    rows_v.at[slot, b], gsem)
                for b in range(KBUF)
            ]

        def scatter_group(slot):
            return [
                pltpu.async_copy(rows_v.at[slot, b],
                                 acc_s.at[didx_v.at[slot * KBUF + b]],
                                 ssem, add=True)
                for b in range(KBUF)
            ]

        def body(p, carry):
            base = ebase + p * (2 * GROUP)

            @pl.when(base < N_EDGES)
            def _():
                pltpu.sync_copy(src_hbm.at[pl.ds(base, 2 * GROUP)], sidx_v)
                pltpu.sync_copy(
                    dst2_hbm.at[pl.ds(wid * NCHUNKS + p * 2 * KBUF, 2 * KBUF)],
                    didx_v)
                for desc in gather_group(0):
                    desc.wait()
                gathers_b = gather_group(1)
                for desc in scatter_group(0):
                    desc.wait()
                for desc in gathers_b:
                    desc.wait()
                for desc in scatter_group(1):
                    desc.wait()

            return carry

        lax.fori_loop(0, NPAIRS, body, 0)

        ones = jnp.ones((L,), jnp.float32)

        def deg_body(g, carry):
            base = ebase + g * DEGB

            @pl.when(base < N_EDGES)
            def _():
                pltpu.sync_copy(dst_hbm.at[pl.ds(base, DEGB)], degidx_v)
                for k in range(DEGB // L):
                    idx = degidx_v[pl.ds(k * L, L)]
                    plsc.addupdate_scatter(deg_v, [idx], ones)

            return carry

        lax.fori_loop(0, NDEGG, deg_body, 0)

        plsc.subcore_barrier()

        pltpu.sync_copy(acc_s.at[pl.ds(zbase, ZROWS)],
                        acc_out.at[pl.ds(cid * NPAD + zbase, ZROWS)])
        pltpu.sync_copy(deg_v, deg_out.at[pl.ds(wid * NPAD, NPAD)])

    return seg_kernel(x, src, dst, dst2, zeros_feat, zeros_deg)


def _tc_degsum_kernel(dp_ref, out_ref):
    out_ref[...] = jnp.maximum(jnp.sum(dp_ref[...], axis=0), 1.0)


def _tc_degsum(deg_parts):
    return pl.pallas_call(
        _tc_degsum_kernel,
        out_shape=jax.ShapeDtypeStruct((NPAD // D, D), jnp.float32),
    )(deg_parts)


ROWS_BLK = 1000


def _tc_combine_kernel(acc_ref, d_ref, x_ref, wl_ref, wr_ref, b_ref, out_ref):
    agg = (acc_ref[0] + acc_ref[1]) / d_ref[...]
    out = (jnp.dot(agg, wl_ref[...], preferred_element_type=jnp.float32)
           + jnp.dot(x_ref[...], wr_ref[...], preferred_element_type=jnp.float32)
           + b_ref[...])
    out_ref[...] = jnp.maximum(out, 0.0)


def _tc_combine(acc, deg, x, w_l, w_r, b):
    grid = (N_NODES // ROWS_BLK,)
    return pl.pallas_call(
        _tc_combine_kernel,
        grid=grid,
        in_specs=[
            pl.BlockSpec((NC, ROWS_BLK, D), lambda i: (0, i, 0)),
            pl.BlockSpec((ROWS_BLK, 1), lambda i: (i, 0)),
            pl.BlockSpec((ROWS_BLK, D), lambda i: (i, 0)),
            pl.BlockSpec((D, D), lambda i: (0, 0)),
            pl.BlockSpec((D, D), lambda i: (0, 0)),
            pl.BlockSpec((1, D), lambda i: (0, 0)),
        ],
        out_specs=pl.BlockSpec((ROWS_BLK, D), lambda i: (i, 0)),
        out_shape=jax.ShapeDtypeStruct((N_NODES, D), jnp.float32),
    )(acc, deg, x, w_l, w_r, b)


def kernel(x, edge_index, W_l, W_r, b):
    src = edge_index[0]
    dst = edge_index[1]
    dst2 = dst.reshape(N_EDGES // CHUNK, CHUNK)
    zeros_feat = jnp.zeros((NPAD, D), jnp.float32)
    zeros_deg = jnp.zeros((NPAD,), jnp.float32)
    acc, deg_parts = _sc_segment_sum(x, src, dst, dst2, zeros_feat, zeros_deg)
    acc = acc.reshape(NC, NPAD, D)
    deg = _tc_degsum(deg_parts.reshape(NW, NPAD // D, D))
    deg = deg.reshape(NPAD, 1)
    return _tc_combine(acc, deg, x, W_l, W_r, b.reshape(1, D))


# speedup vs baseline: 1.0495x; 1.0495x over previous
import functools

import jax
import jax.numpy as jnp
from jax import lax
from jax.experimental import pallas as pl
from jax.experimental.pallas import tpu as pltpu
from jax.experimental.pallas import tpu_sc as plsc

N_NODES = 10000
N_EDGES = 320000
D = 128
L = 16

NC = 2
NS = 16
NW = NC * NS

NPAD = 10240
EDGES_PER_TILE = 10240
E_PAD = NW * EDGES_PER_TILE
CHUNK = 32
KBUF = 8
GROUP = KBUF * CHUNK
NGROUPS = EDGES_PER_TILE // GROUP
NCHUNKS = EDGES_PER_TILE // CHUNK
DEGB = 2048
NDEGG = EDGES_PER_TILE // DEGB
ZROWS = NPAD // NS


def _sc_segment_sum(x, src, dst, dst2, zeros_feat, zeros_deg):
    mesh = plsc.VectorSubcoreMesh(
        core_axis_name="c", subcore_axis_name="s", num_cores=NC, num_subcores=NS
    )

    @functools.partial(
        pl.kernel,
        mesh=mesh,
        compiler_params=pltpu.CompilerParams(needs_layout_passes=False),
        out_type=[
            jax.ShapeDtypeStruct((NC * NPAD, D), jnp.float32),
            jax.ShapeDtypeStruct((NW * NPAD,), jnp.float32),
        ],
        scratch_types=[
            pltpu.VMEM((GROUP,), jnp.int32),
            pltpu.VMEM((KBUF, CHUNK), jnp.int32),
            pltpu.VMEM((KBUF, CHUNK, D), jnp.float32),
            pltpu.VMEM((DEGB,), jnp.int32),
            pltpu.VMEM((NPAD,), jnp.float32),
            pltpu.VMEM_SHARED((NPAD, D), jnp.float32),
            pltpu.SemaphoreType.DMA,
            pltpu.SemaphoreType.DMA,
        ],
    )
    def seg_kernel(x_hbm, src_hbm, dst_hbm, dst2_hbm, zf_hbm, zd_hbm,
                   acc_out, deg_out,
                   sidx_v, didx_v, rows_v, degidx_v, deg_v, acc_s,
                   gsem, ssem):
        cid = lax.axis_index("c")
        sid = lax.axis_index("s")
        wid = sid * NC + cid
        zbase = sid * ZROWS
        ebase = wid * EDGES_PER_TILE

        pltpu.sync_copy(zf_hbm.at[pl.ds(zbase, ZROWS)], acc_s.at[pl.ds(zbase, ZROWS)])
        pltpu.sync_copy(zd_hbm, deg_v)
        plsc.subcore_barrier()

        def body(g, carry):
            base = ebase + g * GROUP

            @pl.when(base < N_EDGES)
            def _():
                pltpu.sync_copy(src_hbm.at[pl.ds(base, GROUP)], sidx_v)
                pltpu.sync_copy(
                    dst2_hbm.at[pl.ds(wid * NCHUNKS + g * KBUF, KBUF)], didx_v)
                gathers = [
                    pltpu.async_copy(
                        x_hbm.at[sidx_v.at[pl.ds(b * CHUNK, CHUNK)]],
                        rows_v.at[b], gsem)
                    for b in range(KBUF)
                ]
                for desc in gathers:
                    desc.wait()
                scatters = [
                    pltpu.async_copy(rows_v.at[b], acc_s.at[didx_v.at[b]],
                                     ssem, add=True)
                    for b in range(KBUF)
                ]
                for desc in scatters:
                    desc.wait()

            return carry

        lax.fori_loop(0, NGROUPS, body, 0)

        ones = jnp.ones((L,), jnp.float32)

        def deg_body(g, carry):
            pltpu.sync_copy(dst_hbm.at[pl.ds(ebase + g * DEGB, DEGB)], degidx_v)
            for k in range(DEGB // L):
                idx = degidx_v[pl.ds(k * L, L)]
                plsc.addupdate_scatter(deg_v, [idx], ones)
            return carry

        lax.fori_loop(0, NDEGG, deg_body, 0)

        plsc.subcore_barrier()

        pltpu.sync_copy(acc_s.at[pl.ds(zbase, ZROWS)],
                        acc_out.at[pl.ds(cid * NPAD + zbase, ZROWS)])
        pltpu.sync_copy(deg_v, deg_out.at[pl.ds(wid * NPAD, NPAD)])

    return seg_kernel(x, src, dst, dst2, zeros_feat, zeros_deg)


def _tc_degsum_kernel(dp_ref, out_ref):
    out_ref[...] = jnp.maximum(jnp.sum(dp_ref[...], axis=0), 1.0)


def _tc_degsum(deg_parts):
    return pl.pallas_call(
        _tc_degsum_kernel,
        out_shape=jax.ShapeDtypeStruct((NPAD // D, D), jnp.float32),
    )(deg_parts)


ROWS_BLK = 1000


def _tc_combine_kernel(acc_ref, d_ref, x_ref, wl_ref, wr_ref, b_ref, out_ref):
    agg = (acc_ref[0] + acc_ref[1]) / d_ref[...]
    out = (jnp.dot(agg, wl_ref[...], preferred_element_type=jnp.float32)
           + jnp.dot(x_ref[...], wr_ref[...], preferred_element_type=jnp.float32)
           + b_ref[...])
    out_ref[...] = jnp.maximum(out, 0.0)


def _tc_combine(acc, deg, x, w_l, w_r, b):
    grid = (N_NODES // ROWS_BLK,)
    return pl.pallas_call(
        _tc_combine_kernel,
        grid=grid,
        in_specs=[
            pl.BlockSpec((NC, ROWS_BLK, D), lambda i: (0, i, 0)),
            pl.BlockSpec((ROWS_BLK, 1), lambda i: (i, 0)),
            pl.BlockSpec((ROWS_BLK, D), lambda i: (i, 0)),
            pl.BlockSpec((D, D), lambda i: (0, 0)),
            pl.BlockSpec((D, D), lambda i: (0, 0)),
            pl.BlockSpec((1, D), lambda i: (0, 0)),
        ],
        out_specs=pl.BlockSpec((ROWS_BLK, D), lambda i: (i, 0)),
        out_shape=jax.ShapeDtypeStruct((N_NODES, D), jnp.float32),
    )(acc, deg, x, w_l, w_r, b)


def kernel(x, edge_index, W_l, W_r, b):
    npad_edges = E_PAD - N_EDGES
    src = jnp.concatenate(
        [edge_index[0], jnp.zeros((npad_edges,), jnp.int32)])
    dst = jnp.concatenate(
        [edge_index[1], jnp.full((npad_edges,), NPAD - 1, jnp.int32)])
    dst2 = dst.reshape(NW * NCHUNKS, CHUNK)
    zeros_feat = jnp.zeros((NPAD, D), jnp.float32)
    zeros_deg = jnp.zeros((NPAD,), jnp.float32)
    acc, deg_parts = _sc_segment_sum(x, src, dst, dst2, zeros_feat, zeros_deg)
    acc = acc.reshape(NC, NPAD, D)
    deg = _tc_degsum(deg_parts.reshape(NW, NPAD // D, D))
    deg = deg.reshape(NPAD, 1)
    return _tc_combine(acc, deg, x, W_l, W_r, b.reshape(1, D))
